# Initial kernel scaffold; baseline (speedup 1.0000x reference)
import jax
import jax.numpy as jnp
from jax.experimental import pallas as pl


def kernel(seq, segment_lab, token_table, pos_table, seg_table):
    raise NotImplementedError("write your pallas kernel here")


# speedup vs baseline: 8.2009x; 8.2009x over previous
import functools

import jax
import jax.numpy as jnp
from jax import lax
from jax.experimental import pallas as pl
from jax.experimental.pallas import tpu as pltpu
from jax.experimental.pallas import tpu_sc as plsc

LANES = 16


def _fuse_body(token_ref, pos_ref, seg_ref, out_ref):
    tp = token_ref[...] + pos_ref[...]
    out_ref[...] = tp[None, :, :] + seg_ref[...][:, None, :]


def _build_fused(token_table, pos_table, seg_table):
    V, D = token_table.shape
    S = seg_table.shape[0]
    BV = 1000
    assert V % BV == 0
    return pl.pallas_call(
        _fuse_body,
        grid=(V // BV,),
        in_specs=[
            pl.BlockSpec((BV, D), lambda i: (i, 0)),
            pl.BlockSpec((BV, D), lambda i: (i, 0)),
            pl.BlockSpec((S, D), lambda i: (0, 0)),
        ],
        out_specs=pl.BlockSpec((S, BV, D), lambda i: (0, i, 0)),
        out_shape=jax.ShapeDtypeStruct((S, V, D), jnp.float32),
    )(token_table, pos_table, seg_table)


@functools.partial(jax.jit, static_argnums=(3,))
def _sc_gather(seq_f, lab_f, fused, V):
    N = seq_f.shape[0]
    D = fused.shape[1]
    info = plsc.get_sparse_core_info()
    NC, NS = info.num_cores, info.num_subcores
    NW = NC * NS
    assert N % NW == 0
    per_w = N // NW
    BLK = 512
    SUB = 128
    assert per_w % BLK == 0 and BLK % SUB == 0
    nblk = per_w // BLK
    mesh = plsc.VectorSubcoreMesh(core_axis_name="c", subcore_axis_name="s")

    @functools.partial(
        pl.kernel,
        mesh=mesh,
        compiler_params=pltpu.CompilerParams(use_tc_tiling_on_sc=False),
        out_type=jax.ShapeDtypeStruct((N, D), jnp.float32),
        scratch_types=[
            pltpu.VMEM((BLK,), jnp.int32),
            pltpu.VMEM((BLK,), jnp.int32),
            pltpu.VMEM((BLK,), jnp.int32),
            pltpu.VMEM((BLK, D), jnp.float32),
            pltpu.SemaphoreType.DMA,
        ],
    )
    def gather(seq_hbm, lab_hbm, table_hbm, out_hbm, seq_v, lab_v, idx_v,
               rows_v, sem):
        wid = lax.axis_index("s") * NC + lax.axis_index("c")

        def blk(b, carry):
            base = wid * per_w + b * BLK
            pltpu.sync_copy(seq_hbm.at[pl.ds(base, BLK)], seq_v)
            pltpu.sync_copy(lab_hbm.at[pl.ds(base, BLK)], lab_v)

            def cidx(i, c):
                sl = pl.ds(i * LANES, LANES)
                idx_v[sl] = lab_v[sl] * V + seq_v[sl]
                return c

            lax.fori_loop(0, BLK // LANES, cidx, 0)
            cps = []
            for j in range(BLK // SUB):
                sl = pl.ds(j * SUB, SUB)
                cps.append(
                    pltpu.async_copy(table_hbm.at[idx_v.at[sl]],
                                     rows_v.at[sl], sem))
            for cp in cps:
                cp.wait()
            pltpu.sync_copy(rows_v, out_hbm.at[pl.ds(base, BLK)])
            return carry

        lax.fori_loop(0, nblk, blk, 0)

    return gather(seq_f, lab_f, fused)


def kernel(seq, segment_lab, token_table, pos_table, seg_table):
    B, L = seq.shape
    V, D = token_table.shape
    S = seg_table.shape[0]
    fused = _build_fused(token_table, pos_table, seg_table)
    fused = fused.reshape(S * V, D)
    out = _sc_gather(seq.reshape(-1), segment_lab.reshape(-1), fused, V)
    return out.reshape(B, L, D)
